# baseline (speedup 1.0000x reference)
import functools

import jax
import jax.numpy as jnp
from jax import lax
from jax.experimental import pallas as pl
from jax.experimental.pallas import tpu as pltpu
from jax.experimental.pallas import tpu_sc as plsc

_N = 10000
_NP = 10240
_E = 320000
_D = 128
_HD = 64
_NC = 2
_NS = 16
_EPT = _E // _NS
_CH = 80
_NCHUNK = _EPT // _CH
_RB = 400
_GRID = _N // _RB
_TROWS = _NP // _NS


def _logits(xb, wsl, wsh, asl, ash, wdl, wdh, adl, adh):
    xs_lo = jnp.dot(xb, wsl, preferred_element_type=jnp.float32)
    xs_hi = jnp.dot(xb, wsh, preferred_element_type=jnp.float32)
    asrc = (jnp.sum(xs_lo * asl, axis=1, keepdims=True)
            + jnp.sum(xs_hi * ash, axis=1, keepdims=True))
    xd_lo = jnp.dot(xb, wdl, preferred_element_type=jnp.float32)
    xd_hi = jnp.dot(xb, wdh, preferred_element_type=jnp.float32)
    adst = (jnp.sum(xd_lo * adl, axis=1, keepdims=True)
            + jnp.sum(xd_hi * adh, axis=1, keepdims=True))
    return xs_lo, xs_hi, asrc, adst


def _stats_update(i, asrc, adst, stats_ref):
    m1 = jnp.max(asrc)
    m2 = jnp.max(adst)
    lane = lax.broadcasted_iota(jnp.int32, (1, _D), 1)
    vec = jnp.where(lane == 0, m1, jnp.where(lane == 1, m2, -jnp.inf))

    @pl.when(i == 0)
    def _():
        stats_ref[...] = vec

    @pl.when(i != 0)
    def _():
        stats_ref[...] = jnp.maximum(stats_ref[...], vec)


def _pre_body(x_ref, wsl_ref, wsh_ref, asl_ref, ash_ref,
              wdl_ref, wdh_ref, adl_ref, adh_ref,
              xs_ref, asrc_ref, adst_ref, stats_ref):
    i = pl.program_id(0)
    xs_lo, xs_hi, asrc, adst = _logits(
        x_ref[...], wsl_ref[...], wsh_ref[...], asl_ref[...], ash_ref[...],
        wdl_ref[...], wdh_ref[...], adl_ref[...], adh_ref[...])
    xs_ref[0] = xs_lo
    xs_ref[1] = xs_hi
    asrc_ref[...] = asrc
    adst_ref[...] = adst
    _stats_update(i, asrc, adst, stats_ref)


_W_SPECS = [
    pl.BlockSpec((_D, _HD), lambda i: (0, 0)),
    pl.BlockSpec((_D, _HD), lambda i: (0, 0)),
    pl.BlockSpec((1, _HD), lambda i: (0, 0)),
    pl.BlockSpec((1, _HD), lambda i: (0, 0)),
    pl.BlockSpec((_D, _HD), lambda i: (0, 0)),
    pl.BlockSpec((_D, _HD), lambda i: (0, 0)),
    pl.BlockSpec((1, _HD), lambda i: (0, 0)),
    pl.BlockSpec((1, _HD), lambda i: (0, 0)),
]


def _split_w(ws, a_s, wd, a_d):
    return (ws[:, :_HD], ws[:, _HD:], a_s[None, :_HD], a_s[None, _HD:],
            wd[:, :_HD], wd[:, _HD:], a_d[None, :_HD], a_d[None, _HD:])

_PRE_OUT_SPECS = [
    pl.BlockSpec((_NC, _RB, _HD), lambda i: (0, i, 0)),
    pl.BlockSpec((_RB, 1), lambda i: (i, 0)),
    pl.BlockSpec((_RB, 1), lambda i: (i, 0)),
    pl.BlockSpec((1, _D), lambda i: (0, 0)),
]

_PRE_OUT_SHAPE = [
    jax.ShapeDtypeStruct((_NC, _N, _HD), jnp.float32),
    jax.ShapeDtypeStruct((_N, 1), jnp.float32),
    jax.ShapeDtypeStruct((_N, 1), jnp.float32),
    jax.ShapeDtypeStruct((1, _D), jnp.float32),
]


def _pre_call(x, ws, a_s, wd, a_d):
    return pl.pallas_call(
        _pre_body,
        grid=(_GRID,),
        in_specs=[pl.BlockSpec((_RB, _D), lambda i: (i, 0))] + _W_SPECS,
        out_specs=_PRE_OUT_SPECS,
        out_shape=_PRE_OUT_SHAPE,
    )(x, *_split_w(ws, a_s, wd, a_d))


def _mid_body(acc_ref, den_ref, b_ref, wsl_ref, wsh_ref, asl_ref, ash_ref,
              wdl_ref, wdh_ref, adl_ref, adh_ref,
              x1_ref, xs_ref, asrc_ref, adst_ref, stats_ref):
    i = pl.program_id(0)
    tot = jnp.concatenate([acc_ref[0], acc_ref[1]], axis=1)
    x1 = jnp.maximum(tot / (den_ref[...] + 1e-16) + b_ref[...], 0.0)
    x1_ref[...] = x1
    xs_lo, xs_hi, asrc, adst = _logits(
        x1, wsl_ref[...], wsh_ref[...], asl_ref[...], ash_ref[...],
        wdl_ref[...], wdh_ref[...], adl_ref[...], adh_ref[...])
    xs_ref[0] = xs_lo
    xs_ref[1] = xs_hi
    asrc_ref[...] = asrc
    adst_ref[...] = adst
    _stats_update(i, asrc, adst, stats_ref)


def _mid_call(acc, den, b, ws, a_s, wd, a_d):
    return pl.pallas_call(
        _mid_body,
        grid=(_GRID,),
        in_specs=[
            pl.BlockSpec((_NC, _RB, _HD), lambda i: (0, i, 0)),
            pl.BlockSpec((_RB, 1), lambda i: (i, 0)),
            pl.BlockSpec((1, _D), lambda i: (0, 0)),
        ] + _W_SPECS,
        out_specs=[pl.BlockSpec((_RB, _D), lambda i: (i, 0))] + _PRE_OUT_SPECS,
        out_shape=([jax.ShapeDtypeStruct((_N, _D), jnp.float32)]
                   + _PRE_OUT_SHAPE),
    )(acc, den, b, *_split_w(ws, a_s, wd, a_d))


def _post_body(x1_ref, acc_ref, den_ref, b_ref, out_ref):
    tot = jnp.concatenate([acc_ref[0], acc_ref[1]], axis=1)
    x2 = tot / (den_ref[...] + 1e-16) + b_ref[...]
    out_ref[...] = jnp.concatenate([x1_ref[...], x2], axis=1)


def _post_call(x1, acc, den, b):
    rb = 400
    return pl.pallas_call(
        _post_body,
        grid=(_N // rb,),
        in_specs=[
            pl.BlockSpec((rb, _D), lambda i: (i, 0)),
            pl.BlockSpec((_NC, rb, _HD), lambda i: (0, i, 0)),
            pl.BlockSpec((rb, 1), lambda i: (i, 0)),
            pl.BlockSpec((1, _D), lambda i: (0, 0)),
        ],
        out_specs=pl.BlockSpec((rb, 2 * _D), lambda i: (i, 0)),
        out_shape=jax.ShapeDtypeStruct((_N, 2 * _D), jnp.float32),
    )(x1, acc, den, b)


@functools.lru_cache(maxsize=1)
def _build_edge_pass():
    mesh = plsc.VectorSubcoreMesh(core_axis_name="c", subcore_axis_name="s",
                                  num_cores=_NC, num_subcores=_NS)

    @functools.partial(
        pl.kernel,
        out_type=(jax.ShapeDtypeStruct((_NC, _NP, _HD), jnp.float32),
                  jax.ShapeDtypeStruct((_NP,), jnp.float32)),
        mesh=mesh,
        compiler_params=pltpu.CompilerParams(needs_layout_passes=False,
                                             use_tc_tiling_on_sc=False),
        scratch_types=(
            pltpu.VMEM((_NP,), jnp.float32),
            pltpu.VMEM((_NP,), jnp.float32),
            pltpu.VMEM((16,), jnp.float32),
            pltpu.VMEM((_NCHUNK + 1, _CH), jnp.int32),
            pltpu.VMEM((_CH,), jnp.int32),
            pltpu.VMEM((_CH,), jnp.int32),
            pltpu.VMEM((_CH,), jnp.int32),
            pltpu.VMEM((_CH,), jnp.int32),
            pltpu.VMEM((_CH,), jnp.int32),
            pltpu.VMEM((_CH,), jnp.float32),
            pltpu.VMEM((_CH, _HD), jnp.float32),
            pltpu.VMEM((_CH, _HD), jnp.float32),
            pltpu.VMEM((_CH, _HD), jnp.float32),
            pltpu.VMEM((_CH, _HD), jnp.float32),
            pltpu.VMEM((_NP,), jnp.float32),
            pltpu.VMEM((_NS, _NP // 64), jnp.float32),
            pltpu.VMEM((_NP // 64,), jnp.float32),
            pltpu.VMEM((160, _HD), jnp.float32),
            pltpu.MemorySpace.VMEM_SHARED((_NP, _HD), jnp.float32),
            pltpu.MemorySpace.VMEM_SHARED((_NS, _NP // 4), jnp.float32),
            pltpu.SemaphoreType.DMA,
            pltpu.SemaphoreType.DMA,
            pltpu.SemaphoreType.DMA,
            pltpu.SemaphoreType.DMA,
        ),
    )
    def edge_pass(asrc_hbm, adst_hbm, shift_hbm, pkidx_hbm,
                  xs_hbm, acc_out, den_out,
                  asrc_v, adst_v, shift_v, pk_v, ia0, ia1, dc0, dc1, pz,
                  ex_v, g0, g1, s0, s1, den_v, red_v, red_out, tmp_v,
                  acc_sh, den_sh, gsem0, gsem1, ssem0, ssem1):
        c = lax.axis_index("c")
        s = lax.axis_index("s")
        base = c * _N

        pltpu.sync_copy(asrc_hbm, asrc_v.at[pl.ds(0, _N)])
        pltpu.sync_copy(adst_hbm, adst_v.at[pl.ds(0, _N)])
        pltpu.sync_copy(shift_hbm, shift_v)
        pltpu.sync_copy(pkidx_hbm.at[s], pk_v.at[pl.ds(0, _NCHUNK)])

        def zden(j, carry):
            den_v[pl.ds(j * 16, 16)] = jnp.zeros((16,), jnp.float32)
            return carry

        lax.fori_loop(0, _NP // 16, zden, 0)

        def ztmp(j, carry):
            for k in range(_HD // 16):
                tmp_v[j, pl.ds(k * 16, 16)] = jnp.zeros((16,), jnp.float32)
            return carry

        lax.fori_loop(0, 160, ztmp, 0)

        for h in range(_TROWS // 160):
            pltpu.sync_copy(tmp_v, acc_sh.at[pl.ds(s * _TROWS + h * 160, 160)])
        plsc.subcore_barrier()

        shift16 = shift_v[...]
        zi16 = jnp.zeros((16,), jnp.int32)
        for g in range(_CH // 16):
            pk_v[_NCHUNK, pl.ds(g * 16, 16)] = zi16
            pz[pl.ds(g * 16, 16)] = zi16
        zf16 = jnp.zeros((16,), jnp.float32)

        def zs(j, carry):
            for k in range(_HD // 16):
                s0[j, pl.ds(k * 16, 16)] = zf16
                s1[j, pl.ds(k * 16, 16)] = zf16
            return carry

        lax.fori_loop(0, _CH, zs, 0)
        pltpu.async_copy(s0, acc_sh.at[pz], ssem0, add=True)
        pltpu.async_copy(s1, acc_sh.at[pz], ssem1, add=True)
        for g in range(_CH // 16):
            p16 = pk_v[0, pl.ds(g * 16, 16)]
            ia0[pl.ds(g * 16, 16)] = (p16 & 0x3FFF) + base
        pltpu.async_copy(xs_hbm.at[ia0], g0, gsem0)

        gbufs, sbufs = (g0, g1), (s0, s1)
        iabs, dcbs = (ia0, ia1), (dc0, dc1)
        gsems, ssems = (gsem0, gsem1), (ssem0, ssem1)

        def step(t, carry):
            for b in range(2):
                ci = 2 * t + b
                gbuf, sbuf = gbufs[b], sbufs[b]
                iab, iao = iabs[b], iabs[1 - b]
                dcb = dcbs[b]
                gsb, gso = gsems[b], gsems[1 - b]
                ssb = ssems[b]
                gob = gbufs[1 - b]
                for g in range(_CH // 16):
                    p16 = pk_v[ci + 1, pl.ds(g * 16, 16)]
                    iao[pl.ds(g * 16, 16)] = (p16 & 0x3FFF) + base
                pltpu.async_copy(xs_hbm.at[iao], gob, gso)
                for g in range(_CH // 16):
                    p16 = pk_v[ci, pl.ds(g * 16, 16)]
                    s16 = p16 & 0x3FFF
                    d16 = p16 >> 14
                    a1 = plsc.load_gather(asrc_v, [s16])
                    a2 = plsc.load_gather(adst_v, [d16])
                    z = a1 + a2
                    z = jnp.maximum(z, 0.2 * z)
                    ex = jnp.exp(z - shift16)
                    ex_v[pl.ds(g * 16, 16)] = ex
                    plsc.addupdate_scatter(den_v, [d16], ex)
                pltpu.make_async_copy(xs_hbm.at[iab], gbuf, gsb).wait()
                pltpu.make_async_copy(sbuf, acc_sh.at[dcb], ssb).wait()
                for g in range(_CH // 16):
                    p16 = pk_v[ci, pl.ds(g * 16, 16)]
                    dcb[pl.ds(g * 16, 16)] = p16 >> 14

                def scale(g, inner):
                    ex16 = ex_v[pl.ds(g * 16, 16)]
                    for l in range(16):
                        j = g * 16 + l
                        wv = jnp.full((16,), ex16[l], jnp.float32)
                        for k in range(_HD // 16):
                            sbuf[j, pl.ds(k * 16, 16)] = (
                                gbuf[j, pl.ds(k * 16, 16)] * wv)
                    return inner

                lax.fori_loop(0, _CH // 16, scale, 0)
                pltpu.async_copy(sbuf, acc_sh.at[dcb], ssb, add=True)
            return carry

        lax.fori_loop(0, _NCHUNK // 2, step, 0)
        pltpu.make_async_copy(xs_hbm.at[ia0], g0, gsem0).wait()
        pltpu.make_async_copy(s0, acc_sh.at[dc0], ssem0).wait()
        pltpu.make_async_copy(s1, acc_sh.at[dc1], ssem1).wait()
        plsc.subcore_barrier()

        @pl.when(c == 0)
        def _():
            half_n = _NP // 4
            seg = _NP // 64
            for half in range(4):
                pltpu.sync_copy(den_v.at[pl.ds(half * half_n, half_n)],
                                den_sh.at[s])
                plsc.subcore_barrier()
                pltpu.sync_copy(den_sh.at[:, pl.ds(s * seg, seg)], red_v)

                def dred(g, carry):
                    acc16 = jnp.zeros((16,), jnp.float32)
                    for r in range(_NS):
                        acc16 = acc16 + red_v[r, pl.ds(g * 16, 16)]
                    red_out[pl.ds(g * 16, 16)] = acc16
                    return carry

                lax.fori_loop(0, seg // 16, dred, 0)
                pltpu.sync_copy(
                    red_out, den_out.at[pl.ds(half * half_n + s * seg, seg)])
                plsc.subcore_barrier()

        for h in range(_TROWS // 160):
            rbase = s * _TROWS + h * 160
            pltpu.sync_copy(acc_sh.at[pl.ds(rbase, 160)], tmp_v)
            pltpu.sync_copy(tmp_v, acc_out.at[c, pl.ds(rbase, 160)])

    return edge_pass


def kernel(x, edge_index, W_src1, W_dst1, att_src1, att_dst1, b1,
           W_src2, W_dst2, att_src2, att_dst2, b2):
    f32 = jnp.float32
    xp = x.astype(f32)
    ei = edge_index.astype(jnp.int32)
    pk = (ei[0] | (ei[1] << 14)).reshape(_NS, _NCHUNK, _CH)

    edge_pass = _build_edge_pass()

    xs1, asrc1, adst1, stats1 = _pre_call(
        xp, W_src1, att_src1, W_dst1, att_dst1)
    shift1 = jnp.maximum(stats1[0, 0] + stats1[0, 1], 0.0)
    acc1, den1 = edge_pass(
        asrc1.reshape(_N), adst1.reshape(_N),
        jnp.full((16,), shift1, f32), pk,
        xs1.reshape(_NC * _N, _HD))

    x1, xs2, asrc2, adst2, stats2 = _mid_call(
        acc1, den1[:, None], b1[None, :], W_src2, att_src2,
        W_dst2, att_dst2)
    shift2 = jnp.maximum(stats2[0, 0] + stats2[0, 1], 0.0)
    acc2, den2 = edge_pass(
        asrc2.reshape(_N), adst2.reshape(_N),
        jnp.full((16,), shift2, f32), pk,
        xs2.reshape(_NC * _N, _HD))

    return _post_call(x1, acc2, den2[:, None], b2[None, :])


# speedup vs baseline: 1.0114x; 1.0114x over previous
import functools

import jax
import jax.numpy as jnp
from jax import lax
from jax.experimental import pallas as pl
from jax.experimental.pallas import tpu as pltpu
from jax.experimental.pallas import tpu_sc as plsc

_N = 10000
_NP = 10240
_E = 320000
_D = 128
_HD = 64
_NC = 2
_NS = 16
_EPT = _E // _NS
_CH = 80
_NCHUNK = _EPT // _CH
_RB = 512
_GRID = _NP // _RB
_TROWS = _NP // _NS


def _logits(xb, wsl, wsh, asl, ash, wdl, wdh, adl, adh):
    xs_lo = jnp.dot(xb, wsl, preferred_element_type=jnp.float32)
    xs_hi = jnp.dot(xb, wsh, preferred_element_type=jnp.float32)
    asrc = (jnp.sum(xs_lo * asl, axis=1, keepdims=True)
            + jnp.sum(xs_hi * ash, axis=1, keepdims=True))
    xd_lo = jnp.dot(xb, wdl, preferred_element_type=jnp.float32)
    xd_hi = jnp.dot(xb, wdh, preferred_element_type=jnp.float32)
    adst = (jnp.sum(xd_lo * adl, axis=1, keepdims=True)
            + jnp.sum(xd_hi * adh, axis=1, keepdims=True))
    return xs_lo, xs_hi, asrc, adst


def _stats_update(i, asrc, adst, stats_ref):
    m1 = jnp.max(asrc)
    m2 = jnp.max(adst)
    lane = lax.broadcasted_iota(jnp.int32, (1, _D), 1)
    vec = jnp.where(lane == 0, m1, jnp.where(lane == 1, m2, -jnp.inf))

    @pl.when(i == 0)
    def _():
        stats_ref[...] = vec

    @pl.when(i != 0)
    def _():
        stats_ref[...] = jnp.maximum(stats_ref[...], vec)


def _pre_body(x_ref, wsl_ref, wsh_ref, asl_ref, ash_ref,
              wdl_ref, wdh_ref, adl_ref, adh_ref,
              xs_ref, asrc_ref, adst_ref, stats_ref):
    i = pl.program_id(0)
    xs_lo, xs_hi, asrc, adst = _logits(
        x_ref[...], wsl_ref[...], wsh_ref[...], asl_ref[...], ash_ref[...],
        wdl_ref[...], wdh_ref[...], adl_ref[...], adh_ref[...])
    xs_ref[0] = xs_lo
    xs_ref[1] = xs_hi
    asrc_ref[...] = asrc
    adst_ref[...] = adst
    _stats_update(i, asrc, adst, stats_ref)


_W_SPECS = [
    pl.BlockSpec((_D, _HD), lambda i: (0, 0)),
    pl.BlockSpec((_D, _HD), lambda i: (0, 0)),
    pl.BlockSpec((1, _HD), lambda i: (0, 0)),
    pl.BlockSpec((1, _HD), lambda i: (0, 0)),
    pl.BlockSpec((_D, _HD), lambda i: (0, 0)),
    pl.BlockSpec((_D, _HD), lambda i: (0, 0)),
    pl.BlockSpec((1, _HD), lambda i: (0, 0)),
    pl.BlockSpec((1, _HD), lambda i: (0, 0)),
]


def _split_w(ws, a_s, wd, a_d):
    return (ws[:, :_HD], ws[:, _HD:], a_s[None, :_HD], a_s[None, _HD:],
            wd[:, :_HD], wd[:, _HD:], a_d[None, :_HD], a_d[None, _HD:])

_PRE_OUT_SPECS = [
    pl.BlockSpec((_NC, _RB, _HD), lambda i: (0, i, 0)),
    pl.BlockSpec((_RB, 1), lambda i: (i, 0)),
    pl.BlockSpec((_RB, 1), lambda i: (i, 0)),
    pl.BlockSpec((1, _D), lambda i: (0, 0)),
]

_PRE_OUT_SHAPE = [
    jax.ShapeDtypeStruct((_NC, _NP, _HD), jnp.float32),
    jax.ShapeDtypeStruct((_NP, 1), jnp.float32),
    jax.ShapeDtypeStruct((_NP, 1), jnp.float32),
    jax.ShapeDtypeStruct((1, _D), jnp.float32),
]


def _pre_call(x, ws, a_s, wd, a_d):
    return pl.pallas_call(
        _pre_body,
        grid=(_GRID,),
        in_specs=[pl.BlockSpec((_RB, _D), lambda i: (i, 0))] + _W_SPECS,
        out_specs=_PRE_OUT_SPECS,
        out_shape=_PRE_OUT_SHAPE,
    )(x, *_split_w(ws, a_s, wd, a_d))


def _mid_body(acc_ref, den_ref, b_ref, wsl_ref, wsh_ref, asl_ref, ash_ref,
              wdl_ref, wdh_ref, adl_ref, adh_ref,
              x1_ref, xs_ref, asrc_ref, adst_ref, stats_ref):
    i = pl.program_id(0)
    tot = jnp.concatenate([acc_ref[0], acc_ref[1]], axis=1)
    x1 = jnp.maximum(tot / (den_ref[...] + 1e-16) + b_ref[...], 0.0)
    x1_ref[...] = x1
    xs_lo, xs_hi, asrc, adst = _logits(
        x1, wsl_ref[...], wsh_ref[...], asl_ref[...], ash_ref[...],
        wdl_ref[...], wdh_ref[...], adl_ref[...], adh_ref[...])
    xs_ref[0] = xs_lo
    xs_ref[1] = xs_hi
    asrc_ref[...] = asrc
    adst_ref[...] = adst
    _stats_update(i, asrc, adst, stats_ref)


def _mid_call(acc, den, b, ws, a_s, wd, a_d):
    return pl.pallas_call(
        _mid_body,
        grid=(_GRID,),
        in_specs=[
            pl.BlockSpec((_NC, _RB, _HD), lambda i: (0, i, 0)),
            pl.BlockSpec((_RB, 1), lambda i: (i, 0)),
            pl.BlockSpec((1, _D), lambda i: (0, 0)),
        ] + _W_SPECS,
        out_specs=[pl.BlockSpec((_RB, _D), lambda i: (i, 0))] + _PRE_OUT_SPECS,
        out_shape=([jax.ShapeDtypeStruct((_NP, _D), jnp.float32)]
                   + _PRE_OUT_SHAPE),
    )(acc, den, b, *_split_w(ws, a_s, wd, a_d))


def _post_body(x1_ref, acc_ref, den_ref, b_ref, out_ref):
    tot = jnp.concatenate([acc_ref[0], acc_ref[1]], axis=1)
    x2 = tot / (den_ref[...] + 1e-16) + b_ref[...]
    out_ref[...] = jnp.concatenate([x1_ref[...], x2], axis=1)


def _post_call(x1, acc, den, b):
    rb = 400
    return pl.pallas_call(
        _post_body,
        grid=(_N // rb,),
        in_specs=[
            pl.BlockSpec((rb, _D), lambda i: (i, 0)),
            pl.BlockSpec((_NC, rb, _HD), lambda i: (0, i, 0)),
            pl.BlockSpec((rb, 1), lambda i: (i, 0)),
            pl.BlockSpec((1, _D), lambda i: (0, 0)),
        ],
        out_specs=pl.BlockSpec((rb, 2 * _D), lambda i: (i, 0)),
        out_shape=jax.ShapeDtypeStruct((_N, 2 * _D), jnp.float32),
    )(x1, acc, den, b)


@functools.lru_cache(maxsize=1)
def _build_edge_pass():
    mesh = plsc.VectorSubcoreMesh(core_axis_name="c", subcore_axis_name="s",
                                  num_cores=_NC, num_subcores=_NS)

    @functools.partial(
        pl.kernel,
        out_type=(jax.ShapeDtypeStruct((_NC, _NP, _HD), jnp.float32),
                  jax.ShapeDtypeStruct((_NP,), jnp.float32)),
        mesh=mesh,
        compiler_params=pltpu.CompilerParams(needs_layout_passes=False,
                                             use_tc_tiling_on_sc=False),
        scratch_types=(
            pltpu.VMEM((_NP,), jnp.float32),
            pltpu.VMEM((_NP,), jnp.float32),
            pltpu.VMEM((16,), jnp.float32),
            pltpu.VMEM((_NCHUNK + 1, _CH), jnp.int32),
            pltpu.VMEM((_CH,), jnp.int32),
            pltpu.VMEM((_CH,), jnp.int32),
            pltpu.VMEM((_CH,), jnp.int32),
            pltpu.VMEM((_CH,), jnp.int32),
            pltpu.VMEM((_CH,), jnp.int32),
            pltpu.VMEM((_CH,), jnp.float32),
            pltpu.VMEM((_CH, _HD), jnp.float32),
            pltpu.VMEM((_CH, _HD), jnp.float32),
            pltpu.VMEM((_CH, _HD), jnp.float32),
            pltpu.VMEM((_CH, _HD), jnp.float32),
            pltpu.VMEM((_NP,), jnp.float32),
            pltpu.VMEM((_NS, _NP // 64), jnp.float32),
            pltpu.VMEM((_NP // 64,), jnp.float32),
            pltpu.VMEM((160, _HD), jnp.float32),
            pltpu.MemorySpace.VMEM_SHARED((_NP, _HD), jnp.float32),
            pltpu.MemorySpace.VMEM_SHARED((_NS, _NP // 4), jnp.float32),
            pltpu.SemaphoreType.DMA,
            pltpu.SemaphoreType.DMA,
            pltpu.SemaphoreType.DMA,
            pltpu.SemaphoreType.DMA,
        ),
    )
    def edge_pass(asrc_hbm, adst_hbm, shift_hbm, pkidx_hbm,
                  xs_hbm, acc_out, den_out,
                  asrc_v, adst_v, shift_v, pk_v, ia0, ia1, dc0, dc1, pz,
                  ex_v, g0, g1, s0, s1, den_v, red_v, red_out, tmp_v,
                  acc_sh, den_sh, gsem0, gsem1, ssem0, ssem1):
        c = lax.axis_index("c")
        s = lax.axis_index("s")
        base = c * _NP

        pltpu.sync_copy(asrc_hbm, asrc_v)
        pltpu.sync_copy(adst_hbm, adst_v)
        pltpu.sync_copy(shift_hbm, shift_v)
        pltpu.sync_copy(pkidx_hbm.at[s], pk_v.at[pl.ds(0, _NCHUNK)])

        def zden(j, carry):
            den_v[pl.ds(j * 16, 16)] = jnp.zeros((16,), jnp.float32)
            return carry

        lax.fori_loop(0, _NP // 16, zden, 0)

        def ztmp(j, carry):
            for k in range(_HD // 16):
                tmp_v[j, pl.ds(k * 16, 16)] = jnp.zeros((16,), jnp.float32)
            return carry

        lax.fori_loop(0, 160, ztmp, 0)

        for h in range(_TROWS // 160):
            pltpu.sync_copy(tmp_v, acc_sh.at[pl.ds(s * _TROWS + h * 160, 160)])
        plsc.subcore_barrier()

        shift16 = shift_v[...]
        zi16 = jnp.zeros((16,), jnp.int32)
        for g in range(_CH // 16):
            pk_v[_NCHUNK, pl.ds(g * 16, 16)] = zi16
            pz[pl.ds(g * 16, 16)] = zi16
        zf16 = jnp.zeros((16,), jnp.float32)

        def zs(j, carry):
            for k in range(_HD // 16):
                s0[j, pl.ds(k * 16, 16)] = zf16
                s1[j, pl.ds(k * 16, 16)] = zf16
            return carry

        lax.fori_loop(0, _CH, zs, 0)
        pltpu.async_copy(s0, acc_sh.at[pz], ssem0, add=True)
        pltpu.async_copy(s1, acc_sh.at[pz], ssem1, add=True)
        for g in range(_CH // 16):
            p16 = pk_v[0, pl.ds(g * 16, 16)]
            ia0[pl.ds(g * 16, 16)] = (p16 & 0x3FFF) + base
        pltpu.async_copy(xs_hbm.at[ia0], g0, gsem0)

        gbufs, sbufs = (g0, g1), (s0, s1)
        iabs, dcbs = (ia0, ia1), (dc0, dc1)
        gsems, ssems = (gsem0, gsem1), (ssem0, ssem1)

        def step(t, carry):
            for b in range(2):
                ci = 2 * t + b
                gbuf, sbuf = gbufs[b], sbufs[b]
                iab, iao = iabs[b], iabs[1 - b]
                dcb = dcbs[b]
                gsb, gso = gsems[b], gsems[1 - b]
                ssb = ssems[b]
                gob = gbufs[1 - b]
                for g in range(_CH // 16):
                    p16 = pk_v[ci + 1, pl.ds(g * 16, 16)]
                    iao[pl.ds(g * 16, 16)] = (p16 & 0x3FFF) + base
                pltpu.async_copy(xs_hbm.at[iao], gob, gso)
                for g in range(_CH // 16):
                    p16 = pk_v[ci, pl.ds(g * 16, 16)]
                    s16 = p16 & 0x3FFF
                    d16 = p16 >> 14
                    a1 = plsc.load_gather(asrc_v, [s16])
                    a2 = plsc.load_gather(adst_v, [d16])
                    z = a1 + a2
                    z = jnp.maximum(z, 0.2 * z)
                    ex = jnp.exp(z - shift16)
                    ex_v[pl.ds(g * 16, 16)] = ex
                    plsc.addupdate_scatter(den_v, [d16], ex)
                pltpu.make_async_copy(xs_hbm.at[iab], gbuf, gsb).wait()
                pltpu.make_async_copy(sbuf, acc_sh.at[dcb], ssb).wait()
                for g in range(_CH // 16):
                    p16 = pk_v[ci, pl.ds(g * 16, 16)]
                    dcb[pl.ds(g * 16, 16)] = p16 >> 14

                def scale(g, inner):
                    ex16 = ex_v[pl.ds(g * 16, 16)]
                    for l in range(16):
                        j = g * 16 + l
                        wv = jnp.full((16,), ex16[l], jnp.float32)
                        for k in range(_HD // 16):
                            sbuf[j, pl.ds(k * 16, 16)] = (
                                gbuf[j, pl.ds(k * 16, 16)] * wv)
                    return inner

                lax.fori_loop(0, _CH // 16, scale, 0)
                pltpu.async_copy(sbuf, acc_sh.at[dcb], ssb, add=True)
            return carry

        lax.fori_loop(0, _NCHUNK // 2, step, 0)
        pltpu.make_async_copy(xs_hbm.at[ia0], g0, gsem0).wait()
        pltpu.make_async_copy(s0, acc_sh.at[dc0], ssem0).wait()
        pltpu.make_async_copy(s1, acc_sh.at[dc1], ssem1).wait()
        plsc.subcore_barrier()

        @pl.when(c == 0)
        def _():
            half_n = _NP // 4
            seg = _NP // 64
            for half in range(4):
                pltpu.sync_copy(den_v.at[pl.ds(half * half_n, half_n)],
                                den_sh.at[s])
                plsc.subcore_barrier()
                pltpu.sync_copy(den_sh.at[:, pl.ds(s * seg, seg)], red_v)

                def dred(g, carry):
                    acc16 = jnp.zeros((16,), jnp.float32)
                    for r in range(_NS):
                        acc16 = acc16 + red_v[r, pl.ds(g * 16, 16)]
                    red_out[pl.ds(g * 16, 16)] = acc16
                    return carry

                lax.fori_loop(0, seg // 16, dred, 0)
                pltpu.sync_copy(
                    red_out, den_out.at[pl.ds(half * half_n + s * seg, seg)])
                plsc.subcore_barrier()

        for h in range(_TROWS // 160):
            rbase = s * _TROWS + h * 160
            pltpu.sync_copy(acc_sh.at[pl.ds(rbase, 160)], tmp_v)
            pltpu.sync_copy(tmp_v, acc_out.at[c, pl.ds(rbase, 160)])

    return edge_pass


def kernel(x, edge_index, W_src1, W_dst1, att_src1, att_dst1, b1,
           W_src2, W_dst2, att_src2, att_dst2, b2):
    f32 = jnp.float32
    xp = jnp.pad(x.astype(f32), ((0, _NP - _N), (0, 0)))
    ei = edge_index.astype(jnp.int32)
    pk = (ei[0] | (ei[1] << 14)).reshape(_NS, _NCHUNK, _CH)

    edge_pass = _build_edge_pass()

    xs1, asrc1, adst1, stats1 = _pre_call(
        xp, W_src1, att_src1, W_dst1, att_dst1)
    shift1 = jnp.maximum(stats1[0, 0] + stats1[0, 1], 0.0)
    acc1, den1 = edge_pass(
        asrc1.reshape(_NP), adst1.reshape(_NP),
        jnp.full((16,), shift1, f32), pk,
        xs1.reshape(_NC * _NP, _HD))

    x1, xs2, asrc2, adst2, stats2 = _mid_call(
        acc1, den1[:, None], b1[None, :], W_src2, att_src2,
        W_dst2, att_dst2)
    shift2 = jnp.maximum(stats2[0, 0] + stats2[0, 1], 0.0)
    acc2, den2 = edge_pass(
        asrc2.reshape(_NP), adst2.reshape(_NP),
        jnp.full((16,), shift2, f32), pk,
        xs2.reshape(_NC * _NP, _HD))

    return _post_call(x1, acc2, den2[:, None], b2[None, :])
